# baseline (speedup 1.0000x reference)
import functools

import jax
import jax.numpy as jnp
from jax.experimental import pallas as pl
from jax.experimental.pallas import tpu as pltpu

_B, _S, _D, _E, _DFF, _TOPK = 1, 2048, 1024, 8, 2048, 2
_C1 = 256
_NF1 = _DFF // _C1
_C2 = 128
_ND2 = _D // _C2


def _route_kernel(x_ref, wg_ref, logits_ref, wts_ref, idx_ref):
    x = x_ref[...]
    r = jnp.dot(x, wg_ref[...], preferred_element_type=jnp.float32)
    logits = jnp.mean(r, axis=0, keepdims=True)
    logits_ref[...] = logits
    m = jnp.max(logits)
    ex = jnp.exp(logits - m)
    p = ex / jnp.sum(ex)
    i1 = jnp.argmax(p)
    w1 = jnp.max(p)
    iota = jax.lax.broadcasted_iota(jnp.int32, (1, _E), 1)
    p2 = jnp.where(iota == i1, -jnp.inf, p)
    i2 = jnp.argmax(p2)
    w2 = jnp.max(p2)
    wts_ref[...] = jnp.concatenate(
        [w1.reshape(1, 1), w2.reshape(1, 1)], axis=1)
    idx_ref[...] = jnp.concatenate(
        [i1.astype(jnp.int32).reshape(1, 1), i2.astype(jnp.int32).reshape(1, 1)],
        axis=1)


def _ffn_kernel(idx_sm, wts_sm, x_ref, w1a_ref, w1b_ref, b1a_ref, b1b_ref,
                w2a_ref, w2b_ref, b2a_ref, b2b_ref, out_ref,
                ha_ref, hb_ref):
    i = pl.program_id(0)

    @pl.when(i < _NF1)
    def _():
        x = x_ref[...]
        ha = jnp.dot(x, w1a_ref[0], preferred_element_type=jnp.float32)
        hb = jnp.dot(x, w1b_ref[0], preferred_element_type=jnp.float32)
        c = i * _C1
        ha_ref[:, pl.ds(c, _C1)] = jax.nn.gelu(ha + b1a_ref[0])
        hb_ref[:, pl.ds(c, _C1)] = jax.nn.gelu(hb + b1b_ref[0])

    @pl.when(i >= _NF1)
    def _():
        w0 = wts_sm[0]
        w1 = wts_sm[1]
        oa = jnp.dot(ha_ref[...], w2a_ref[0], preferred_element_type=jnp.float32)
        ob = jnp.dot(hb_ref[...], w2b_ref[0], preferred_element_type=jnp.float32)
        out_ref[...] = w0 * oa + w1 * ob + (w0 * b2a_ref[0] + w1 * b2b_ref[0])


@jax.jit
def kernel(hidden_states, W_gate, W1, b1, W2, b2):
    x2 = hidden_states.reshape(_S, _D)

    logits, wts, idx = pl.pallas_call(
        _route_kernel,
        out_shape=(
            jax.ShapeDtypeStruct((1, _E), jnp.float32),
            jax.ShapeDtypeStruct((1, _TOPK), jnp.float32),
            jax.ShapeDtypeStruct((1, _TOPK), jnp.int32),
        ),
    )(x2, W_gate)

    def _f1(i):
        return jnp.minimum(i, _NF1 - 1)

    def _f2(i):
        return jnp.clip(i - _NF1, 0, _ND2 - 1)

    grid_spec = pltpu.PrefetchScalarGridSpec(
        num_scalar_prefetch=2,
        grid=(_NF1 + _ND2,),
        in_specs=[
            pl.BlockSpec((_S, _D), lambda i, idx_s, wts_s: (0, 0)),
            pl.BlockSpec((1, _D, _C1),
                         lambda i, idx_s, wts_s: (idx_s[0], 0, _f1(i))),
            pl.BlockSpec((1, _D, _C1),
                         lambda i, idx_s, wts_s: (idx_s[1], 0, _f1(i))),
            pl.BlockSpec((1, 1, _C1),
                         lambda i, idx_s, wts_s: (idx_s[0], 0, _f1(i))),
            pl.BlockSpec((1, 1, _C1),
                         lambda i, idx_s, wts_s: (idx_s[1], 0, _f1(i))),
            pl.BlockSpec((1, _DFF, _C2),
                         lambda i, idx_s, wts_s: (idx_s[0], 0, _f2(i))),
            pl.BlockSpec((1, _DFF, _C2),
                         lambda i, idx_s, wts_s: (idx_s[1], 0, _f2(i))),
            pl.BlockSpec((1, 1, _C2),
                         lambda i, idx_s, wts_s: (idx_s[0], 0, _f2(i))),
            pl.BlockSpec((1, 1, _C2),
                         lambda i, idx_s, wts_s: (idx_s[1], 0, _f2(i))),
        ],
        out_specs=pl.BlockSpec((_S, _C2),
                               lambda i, idx_s, wts_s: (0, _f2(i))),
        scratch_shapes=[
            pltpu.VMEM((_S, _DFF), jnp.float32),
            pltpu.VMEM((_S, _DFF), jnp.float32),
        ],
    )
    out = pl.pallas_call(
        _ffn_kernel,
        grid_spec=grid_spec,
        out_shape=jax.ShapeDtypeStruct((_S, _D), jnp.float32),
        compiler_params=pltpu.CompilerParams(
            dimension_semantics=("arbitrary",)),
    )(idx.reshape(_TOPK), wts.reshape(_TOPK), x2, W1, W1,
      b1.reshape(_E, 1, _DFF), b1.reshape(_E, 1, _DFF),
      W2, W2, b2.reshape(_E, 1, _D), b2.reshape(_E, 1, _D))

    return (out.reshape(_B, _S, _D), logits)


# speedup vs baseline: 1.2650x; 1.2650x over previous
import functools

import jax
import jax.numpy as jnp
from jax.experimental import pallas as pl
from jax.experimental.pallas import tpu as pltpu

_B, _S, _D, _E, _DFF, _TOPK = 1, 2048, 1024, 8, 2048, 2
_TS = 512
_NS = _S // _TS


def _route_kernel(x_ref, wg_ref, logits_ref, wts_ref, idx_ref):
    x = x_ref[...]
    r = jnp.dot(x, wg_ref[...], preferred_element_type=jnp.float32)
    logits = jnp.mean(r, axis=0, keepdims=True)
    logits_ref[...] = logits
    m = jnp.max(logits)
    ex = jnp.exp(logits - m)
    p = ex / jnp.sum(ex)
    i1 = jnp.argmax(p)
    w1 = jnp.max(p)
    iota = jax.lax.broadcasted_iota(jnp.int32, (1, _E), 1)
    p2 = jnp.where(iota == i1, -jnp.inf, p)
    i2 = jnp.argmax(p2)
    w2 = jnp.max(p2)
    wts_ref[...] = jnp.concatenate(
        [w1.reshape(1, 1), w2.reshape(1, 1)], axis=1)
    idx_ref[...] = jnp.concatenate(
        [i1.astype(jnp.int32).reshape(1, 1), i2.astype(jnp.int32).reshape(1, 1)],
        axis=1)


def _ffn_kernel(idx_sm, wts_sm, x_ref, w1_ref, b1_ref, w2_ref, b2_ref,
                out_ref, w1bf_ref, w2bf_ref):
    k = pl.program_id(0)
    si = pl.program_id(1)

    @pl.when(si == 0)
    def _():
        w1bf_ref[...] = w1_ref[0].astype(jnp.bfloat16)
        w2bf_ref[...] = w2_ref[0].astype(jnp.bfloat16)

    x = x_ref[...].astype(jnp.bfloat16)
    h = jnp.dot(x, w1bf_ref[...], preferred_element_type=jnp.float32)
    h = jax.nn.gelu(h + b1_ref[0])
    o = jnp.dot(h.astype(jnp.bfloat16), w2bf_ref[...],
                preferred_element_type=jnp.float32)
    contrib = wts_sm[k] * (o + b2_ref[0])

    @pl.when(k == 0)
    def _():
        out_ref[pl.ds(si * _TS, _TS), :] = contrib

    @pl.when(k > 0)
    def _():
        out_ref[pl.ds(si * _TS, _TS), :] = (
            out_ref[pl.ds(si * _TS, _TS), :] + contrib)


@jax.jit
def kernel(hidden_states, W_gate, W1, b1, W2, b2):
    x2 = hidden_states.reshape(_S, _D)

    logits, wts, idx = pl.pallas_call(
        _route_kernel,
        out_shape=(
            jax.ShapeDtypeStruct((1, _E), jnp.float32),
            jax.ShapeDtypeStruct((1, _TOPK), jnp.float32),
            jax.ShapeDtypeStruct((1, _TOPK), jnp.int32),
        ),
    )(x2, W_gate)

    grid_spec = pltpu.PrefetchScalarGridSpec(
        num_scalar_prefetch=2,
        grid=(_TOPK, _NS),
        in_specs=[
            pl.BlockSpec((_TS, _D), lambda k, si, idx_s, wts_s: (si, 0)),
            pl.BlockSpec((1, _D, _DFF),
                         lambda k, si, idx_s, wts_s: (idx_s[k], 0, 0)),
            pl.BlockSpec((1, 1, _DFF),
                         lambda k, si, idx_s, wts_s: (idx_s[k], 0, 0)),
            pl.BlockSpec((1, _DFF, _D),
                         lambda k, si, idx_s, wts_s: (idx_s[k], 0, 0)),
            pl.BlockSpec((1, 1, _D),
                         lambda k, si, idx_s, wts_s: (idx_s[k], 0, 0)),
        ],
        out_specs=pl.BlockSpec((_S, _D), lambda k, si, idx_s, wts_s: (0, 0)),
        scratch_shapes=[
            pltpu.VMEM((_D, _DFF), jnp.bfloat16),
            pltpu.VMEM((_DFF, _D), jnp.bfloat16),
        ],
    )
    out = pl.pallas_call(
        _ffn_kernel,
        grid_spec=grid_spec,
        out_shape=jax.ShapeDtypeStruct((_S, _D), jnp.float32),
        compiler_params=pltpu.CompilerParams(
            dimension_semantics=("arbitrary", "arbitrary")),
    )(idx.reshape(_TOPK), wts.reshape(_TOPK), x2, W1,
      b1.reshape(_E, 1, _DFF), W2, b2.reshape(_E, 1, _D))

    return (out.reshape(_B, _S, _D), logits)
